# Initial kernel scaffold; baseline (speedup 1.0000x reference)
import functools

import jax
import jax.numpy as jnp
from jax import lax
from jax.experimental import pallas as pl
from jax.experimental.pallas import tpu as pltpu, tpu_sc as plsc

N = 10000
D = 128
E = 320000

NC = 2
NS = 16
NW = NC * NS
CHB = 128
NCH = 80
EPAD = NW * NCH * CHB
N_ACC = 10240
ROWS_PER_TILE_Z = N_ACC // NS
ROWS_PER_TILE_O = N // NS
OCH = 125


def _xpw_body(x_ref, p_ref, w_ref, o_ref):
    o_ref[...] = jnp.dot(x_ref[...] * p_ref[...], w_ref[...],
                         preferred_element_type=jnp.float32)


def _xpw(x, p2, W):
    blk = 1000
    return pl.pallas_call(
        _xpw_body,
        grid=(N // blk,),
        in_specs=[
            pl.BlockSpec((blk, D), lambda i: (i, 0)),
            pl.BlockSpec((blk, 1), lambda i: (i, 0)),
            pl.BlockSpec((D, D), lambda i: (0, 0)),
        ],
        out_specs=pl.BlockSpec((blk, D), lambda i: (i, 0)),
        out_shape=jax.ShapeDtypeStruct((N, D), jnp.float32),
    )(x, p2, W)


def _sc_body(y_hbm, src_hbm, dst_hbm, zeros_hbm, out_hbm,
             src_v, dst_v, buf0, buf1, acc, sem0, sem1):
    c = lax.axis_index("c")
    s = lax.axis_index("s")
    w = s * NC + c

    pltpu.sync_copy(src_hbm.at[w], src_v)
    pltpu.sync_copy(dst_hbm.at[w], dst_v)

    pltpu.sync_copy(zeros_hbm, buf0)
    for q in range(ROWS_PER_TILE_Z // CHB):
        pltpu.sync_copy(buf0, acc.at[pl.ds(s * ROWS_PER_TILE_Z + q * CHB, CHB)])
    plsc.subcore_barrier()

    def body(g, carry):
        for bi, (buf, sem) in enumerate(((buf0, sem0), (buf1, sem1))):
            j = g * 2 + bi
            pltpu.async_copy(y_hbm.at[src_v.at[j]], buf, sem).wait()
            pltpu.sync_copy(buf, acc.at[dst_v.at[j]], add=True)
        return carry

    lax.fori_loop(0, NCH // 2, body, 0)
    plsc.subcore_barrier()

    for q in range(ROWS_PER_TILE_O // OCH):
        r0 = s * ROWS_PER_TILE_O + q * OCH
        pltpu.sync_copy(acc.at[pl.ds(r0, OCH)], buf0.at[pl.ds(0, OCH)])
        pltpu.sync_copy(buf0.at[pl.ds(0, OCH)], out_hbm.at[c].at[pl.ds(r0, OCH)])


def _sc_agg(y, src_r, dst_r, zeros):
    mesh = plsc.VectorSubcoreMesh(core_axis_name="c", subcore_axis_name="s")
    k = pl.kernel(
        _sc_body,
        out_type=jax.ShapeDtypeStruct((NC, N, D), jnp.float32),
        mesh=mesh,
        scratch_types=[
            pltpu.VMEM((NCH, CHB), jnp.int32),
            pltpu.VMEM((NCH, CHB), jnp.int32),
            pltpu.VMEM((CHB, D), jnp.float32),
            pltpu.VMEM((CHB, D), jnp.float32),
            pltpu.VMEM_SHARED((N_ACC, D), jnp.float32),
            pltpu.SemaphoreType.DMA,
            pltpu.SemaphoreType.DMA,
        ],
    )
    return k(y, src_r, dst_r, zeros)


def _fin_body(a_ref, b_ref, bias_ref, o_ref):
    o_ref[...] = a_ref[...] + b_ref[...] + bias_ref[...]


def _fin(p0, p1, bias2):
    blk = 1000
    return pl.pallas_call(
        _fin_body,
        grid=(N // blk,),
        in_specs=[
            pl.BlockSpec((blk, D), lambda i: (i, 0)),
            pl.BlockSpec((blk, D), lambda i: (i, 0)),
            pl.BlockSpec((1, D), lambda i: (0, 0)),
        ],
        out_specs=pl.BlockSpec((blk, D), lambda i: (i, 0)),
        out_shape=jax.ShapeDtypeStruct((N, D), jnp.float32),
    )(p0, p1, bias2)


@jax.jit
def kernel(x, edge_index, p, W, b):
    y = _xpw(x, p.reshape(N, 1), W)

    pad = EPAD - E
    src = jnp.concatenate([edge_index[0], jnp.zeros((pad,), jnp.int32)])
    dst = jnp.concatenate([edge_index[1], jnp.full((pad,), N, jnp.int32)])
    src_r = src.reshape(NW, NCH, CHB)
    dst_r = dst.reshape(NW, NCH, CHB)
    zeros = jnp.zeros((CHB, D), jnp.float32)

    parts = _sc_agg(y, src_r, dst_r, zeros)
    return _fin(parts[0], parts[1], b.reshape(1, D))


# speedup vs baseline: 3.5344x; 3.5344x over previous
import functools

import jax
import jax.numpy as jnp
from jax import lax
from jax.experimental import pallas as pl
from jax.experimental.pallas import tpu as pltpu, tpu_sc as plsc

N = 10000
D = 128
E = 320000

NC = 2
NS = 16
NW = NC * NS
CHB = 128
NCH = 80
EPAD = NW * NCH * CHB
N_ACC = 10240
ROWS_PER_TILE_Z = N_ACC // NS


def _xpw_body(x_ref, p_ref, w_ref, o_ref):
    o_ref[...] = jnp.dot(x_ref[...] * p_ref[...], w_ref[...],
                         preferred_element_type=jnp.float32)


def _xpw(x, p2, W):
    blk = 1000
    return pl.pallas_call(
        _xpw_body,
        grid=(N // blk,),
        in_specs=[
            pl.BlockSpec((blk, D), lambda i: (i, 0)),
            pl.BlockSpec((blk, 1), lambda i: (i, 0)),
            pl.BlockSpec((D, D), lambda i: (0, 0)),
        ],
        out_specs=pl.BlockSpec((blk, D), lambda i: (i, 0)),
        out_shape=jax.ShapeDtypeStruct((N, D), jnp.float32),
    )(x, p2, W)


def _sc_body(y_hbm, src_hbm, dst_hbm, zeros_hbm, out_hbm,
             src_v, dst_v, buf0, acc, sem0):
    c = lax.axis_index("c")
    s = lax.axis_index("s")
    w = s * NC + c

    pltpu.sync_copy(src_hbm.at[w], src_v)
    pltpu.sync_copy(dst_hbm.at[w], dst_v)

    pltpu.sync_copy(zeros_hbm, buf0)
    for q in range(ROWS_PER_TILE_Z // CHB):
        pltpu.sync_copy(buf0, acc.at[pl.ds(s * ROWS_PER_TILE_Z + q * CHB, CHB)])
    plsc.subcore_barrier()

    def body(j, carry):
        pltpu.async_copy(y_hbm.at[src_v.at[j]], buf0, sem0).wait()
        pltpu.sync_copy(buf0, acc.at[dst_v.at[j]], add=True)
        return carry

    lax.fori_loop(0, NCH, body, 0)
    plsc.subcore_barrier()

    for q in range(ROWS_PER_TILE_Z // CHB):
        r0 = s * ROWS_PER_TILE_Z + q * CHB
        pltpu.sync_copy(acc.at[pl.ds(r0, CHB)], buf0)
        pltpu.sync_copy(buf0, out_hbm.at[c].at[pl.ds(r0, CHB)])


def _sc_agg(y, src_r, dst_r, zeros):
    mesh = plsc.VectorSubcoreMesh(core_axis_name="c", subcore_axis_name="s")
    k = pl.kernel(
        _sc_body,
        out_type=jax.ShapeDtypeStruct((NC, N_ACC, D), jnp.float32),
        mesh=mesh,
        scratch_types=[
            pltpu.VMEM((NCH, CHB), jnp.int32),
            pltpu.VMEM((NCH, CHB), jnp.int32),
            pltpu.VMEM((CHB, D), jnp.float32),
            pltpu.VMEM_SHARED((N_ACC, D), jnp.float32),
            pltpu.SemaphoreType.DMA,
        ],
    )
    return k(y, src_r, dst_r, zeros)


def _fin_body(a_ref, b_ref, bias_ref, o_ref):
    o_ref[...] = a_ref[...] + b_ref[...] + bias_ref[...]


def _fin(p0, p1, bias2):
    blk = 1000
    return pl.pallas_call(
        _fin_body,
        grid=(N // blk,),
        in_specs=[
            pl.BlockSpec((blk, D), lambda i: (i, 0)),
            pl.BlockSpec((blk, D), lambda i: (i, 0)),
            pl.BlockSpec((1, D), lambda i: (0, 0)),
        ],
        out_specs=pl.BlockSpec((blk, D), lambda i: (i, 0)),
        out_shape=jax.ShapeDtypeStruct((N, D), jnp.float32),
    )(p0, p1, bias2)


@jax.jit
def kernel(x, edge_index, p, W, b):
    y = _xpw(x, p.reshape(N, 1), W)

    pad = EPAD - E
    src = jnp.concatenate([edge_index[0], jnp.zeros((pad,), jnp.int32)])
    dst = jnp.concatenate([edge_index[1], jnp.full((pad,), N, jnp.int32)])
    src_r = src.reshape(NW, NCH, CHB)
    dst_r = dst.reshape(NW, NCH, CHB)
    zeros = jnp.zeros((CHB, D), jnp.float32)

    parts = _sc_agg(y, src_r, dst_r, zeros)
    return _fin(parts[0], parts[1], b.reshape(1, D))
